# Initial kernel scaffold; baseline (speedup 1.0000x reference)
import jax
import jax.numpy as jnp
from jax.experimental import pallas as pl


def kernel(pos, edge_index, nbr_shift, edge_attr, triplet_attr, triplet_index, We0, be0, We1, be1, We2, be2, Weo, beo, Wt0, bt0, Wt1, bt1, Wt2, bt2, Wto, bto):
    raise NotImplementedError("write your pallas kernel here")


# speedup vs baseline: 2.0710x; 2.0710x over previous
import functools

import jax
import jax.numpy as jnp
import numpy as np
from jax.experimental import pallas as pl
from jax.experimental.pallas import tpu as pltpu

N = 50000
E = 800000
T = 1600000
D = 16
PACK = 16
DBIG = D * PACK
CUTOFF2 = 36.0


def _block_diag(W, pack):
    D_, Do = W.shape
    out = jnp.zeros((pack, D_, pack, Do), W.dtype)
    idx = jnp.arange(pack)
    out = out.at[idx, :, idx, :].set(jnp.broadcast_to(W, (pack, D_, Do)))
    return out.reshape(pack * D_, pack * Do)


def _softplus(x):
    return jnp.maximum(x, 0.0) + jnp.log1p(jnp.exp(-jnp.abs(x)))


def _mlp_body(x_ref, w0, b0, w1, b1, w2, b2, wo, bo, out_ref):
    h = x_ref[...]
    h = _softplus(jnp.dot(h, w0[...], preferred_element_type=jnp.float32) + b0[...])
    h = _softplus(jnp.dot(h, w1[...], preferred_element_type=jnp.float32) + b1[...])
    h = _softplus(jnp.dot(h, w2[...], preferred_element_type=jnp.float32) + b2[...])
    out_ref[...] = jnp.dot(h, wo[...], preferred_element_type=jnp.float32) + bo[...]


def _packed_mlp(x, W0, b0, W1, b1, W2, b2, Wo, bo, rows_blk):
    R = x.shape[0]
    grid = (R // rows_blk,)
    wspec16 = pl.BlockSpec((DBIG, DBIG), lambda i: (0, 0))
    wspec_o = pl.BlockSpec((DBIG, PACK), lambda i: (0, 0))
    bspec = pl.BlockSpec((1, DBIG), lambda i: (0, 0))
    bspec_o = pl.BlockSpec((1, PACK), lambda i: (0, 0))
    return pl.pallas_call(
        _mlp_body,
        grid=grid,
        in_specs=[
            pl.BlockSpec((rows_blk, DBIG), lambda i: (i, 0)),
            wspec16, bspec, wspec16, bspec, wspec16, bspec, wspec_o, bspec_o,
        ],
        out_specs=pl.BlockSpec((rows_blk, PACK), lambda i: (i, 0)),
        out_shape=jax.ShapeDtypeStruct((R, PACK), jnp.float32),
    )(x, W0, b0, W1, b1, W2, b2, Wo, bo)


def kernel(pos, edge_index, nbr_shift, edge_attr, triplet_attr, triplet_index,
           We0, be0, We1, be1, We2, be2, Weo, beo,
           Wt0, bt0, Wt1, bt1, Wt2, bt2, Wto, bto):
    We0b, We1b, We2b = (_block_diag(W, PACK) for W in (We0, We1, We2))
    Wt0b, Wt1b, Wt2b = (_block_diag(W, PACK) for W in (Wt0, Wt1, Wt2))
    Weob = _block_diag(Weo, PACK)
    Wtob = _block_diag(Wto, PACK)
    be0b, be1b, be2b = (jnp.tile(b, PACK)[None, :] for b in (be0, be1, be2))
    bt0b, bt1b, bt2b = (jnp.tile(b, PACK)[None, :] for b in (bt0, bt1, bt2))
    beob = jnp.tile(beo, PACK)[None, :]
    btob = jnp.tile(bto, PACK)[None, :]

    xe = edge_attr.reshape(E // PACK, DBIG)
    xt = triplet_attr.reshape(T // PACK, DBIG)
    fe = _packed_mlp(xe, We0b, be0b, We1b, be1b, We2b, be2b, Weob, beob,
                     rows_blk=1000).reshape(E)
    ft = _packed_mlp(xt, Wt0b, bt0b, Wt1b, bt1b, Wt2b, bt2b, Wtob, btob,
                     rows_blk=1000).reshape(T)

    j = edge_index[0]
    i = edge_index[1]
    edge_dir = pos[i] + nbr_shift - pos[j]
    d2 = jnp.sum(edge_dir ** 2, axis=-1)
    inv_len = jax.lax.rsqrt(d2)
    ndir = edge_dir * inv_len[:, None]
    dyad_ji_ji = (ndir[:, :, None] * ndir[:, None, :]).reshape(-1, 9)
    temp_sym = fe[:, None] * dyad_ji_ji
    born_sym = jax.ops.segment_sum(temp_sym, i, num_segments=N)

    idx_j = triplet_index[1]
    idx_kj = triplet_index[3]
    idx_ji = triplet_index[4]
    mdir = ndir * (d2 < CUTOFF2).astype(jnp.float32)[:, None]
    a = mdir[idx_kj]
    b = mdir[idx_ji]
    dyad_kj_ji = (a[:, :, None] * b[:, None, :]).reshape(-1, 9)
    temp_cross = ft[:, None] * dyad_kj_ji
    born_cross = jax.ops.segment_sum(temp_cross, idx_j, num_segments=N)
    return born_sym + born_cross


# speedup vs baseline: 8.5787x; 4.1422x over previous
import functools

import jax
import jax.numpy as jnp
import numpy as np
from jax import lax
from jax.experimental import pallas as pl
from jax.experimental.pallas import tpu as pltpu
from jax.experimental.pallas import tpu_sc as plsc

N = 50000
E = 800000
T = 1600000
D = 16
PACK = 16
DBIG = D * PACK
CUTOFF2 = 36.0

NC = 2
NS = 16
NW = NC * NS
NPAD = 50048
ROWS_PT = NPAD // NS

E_PT = E // NW
T_PT = T // NW
KG = 1000
KS = 1000
KC = 1000

_SC_PARAMS = pltpu.CompilerParams(use_tc_tiling_on_sc=False)

_PU = np.zeros((4, 16), np.float32)
_PW = np.zeros((4, 16), np.float32)
for _c in range(3):
    _PU[_c, 4 * _c:4 * _c + 4] = 1.0
    _PW[_c, _c::4] = 1.0
_DYAD_LANES = np.array([0, 1, 2, 4, 5, 6, 8, 9, 10], np.int32)


def _block_diag(W, pack):
    D_, Do = W.shape
    out = jnp.zeros((pack, D_, pack, Do), W.dtype)
    idx = jnp.arange(pack)
    out = out.at[idx, :, idx, :].set(jnp.broadcast_to(W, (pack, D_, Do)))
    return out.reshape(pack * D_, pack * Do)


def _softplus(x):
    return jnp.maximum(x, 0.0) + jnp.log1p(jnp.exp(-jnp.abs(x)))


def _mlp_body(x_ref, w0, b0, w1, b1, w2, b2, wo, bo, out_ref):
    h = x_ref[...]
    h = _softplus(jnp.dot(h, w0[...], preferred_element_type=jnp.float32) + b0[...])
    h = _softplus(jnp.dot(h, w1[...], preferred_element_type=jnp.float32) + b1[...])
    h = _softplus(jnp.dot(h, w2[...], preferred_element_type=jnp.float32) + b2[...])
    out_ref[...] = jnp.dot(h, wo[...], preferred_element_type=jnp.float32) + bo[...]


def _packed_mlp(x, W0, b0, W1, b1, W2, b2, Wo, bo, rows_blk):
    R = x.shape[0]
    grid = (R // rows_blk,)
    wspec16 = pl.BlockSpec((DBIG, DBIG), lambda i: (0, 0))
    wspec_o = pl.BlockSpec((DBIG, PACK), lambda i: (0, 0))
    bspec = pl.BlockSpec((1, DBIG), lambda i: (0, 0))
    bspec_o = pl.BlockSpec((1, PACK), lambda i: (0, 0))
    return pl.pallas_call(
        _mlp_body,
        grid=grid,
        in_specs=[
            pl.BlockSpec((rows_blk, DBIG), lambda i: (i, 0)),
            wspec16, bspec, wspec16, bspec, wspec16, bspec, wspec_o, bspec_o,
        ],
        out_specs=pl.BlockSpec((rows_blk, PACK), lambda i: (i, 0)),
        out_shape=jax.ShapeDtypeStruct((R, PACK), jnp.float32),
    )(x, W0, b0, W1, b1, W2, b2, Wo, bo)


def _geo_body(a16, b16, sh4, fe, pu_ref, pw_ref, ts16, mdu, mdw):
    pu = pu_ref[...]
    pw = pw_ref[...]
    d = a16[..., :4] - b16[..., :4] + sh4[...]
    d2 = jnp.sum(d * d, axis=1, keepdims=True)
    ir = lax.rsqrt(d2)
    n4 = d * ir
    hp = lax.Precision.HIGHEST
    u16 = jnp.dot(n4, pu, preferred_element_type=jnp.float32, precision=hp)
    w16 = jnp.dot(n4, pw, preferred_element_type=jnp.float32, precision=hp)
    ts16[...] = fe[...] * u16 * w16
    m = (d2 < CUTOFF2).astype(jnp.float32)
    n4m = n4 * m
    mdu[...] = jnp.dot(n4m, pu, preferred_element_type=jnp.float32, precision=hp)
    mdw[...] = jnp.dot(n4m, pw, preferred_element_type=jnp.float32, precision=hp)


def _geometry(a16, b16, sh4, fe, rows_blk=4000):
    grid = (E // rows_blk,)
    spec4 = pl.BlockSpec((rows_blk, 4), lambda i: (i, 0))
    spec16i = pl.BlockSpec((rows_blk, 16), lambda i: (i, 0))
    spec1 = pl.BlockSpec((rows_blk, 1), lambda i: (i, 0))
    spec16 = pl.BlockSpec((rows_blk, 16), lambda i: (i, 0))
    specp = pl.BlockSpec((4, 16), lambda i: (0, 0))
    shp = jax.ShapeDtypeStruct((E, 16), jnp.float32)
    return pl.pallas_call(
        _geo_body,
        grid=grid,
        in_specs=[spec16i, spec16i, spec4, spec1, specp, specp],
        out_specs=[spec16, spec16, spec16],
        out_shape=[shp, shp, shp],
    )(a16, b16, sh4, fe, jnp.asarray(_PU), jnp.asarray(_PW))


def _gather_body(pos16, iidx, jidx, a16_out, b16_out, i_v, j_v, a_v, b_v):
    c = lax.axis_index("c")
    s = lax.axis_index("s")
    wid = s * NC + c

    def chunk(k, _):
        base = wid * E_PT + k * KG
        pltpu.sync_copy(iidx.at[pl.ds(base, KG)], i_v)
        pltpu.sync_copy(jidx.at[pl.ds(base, KG)], j_v)
        pltpu.sync_copy(pos16.at[i_v], a_v)
        pltpu.sync_copy(pos16.at[j_v], b_v)
        pltpu.sync_copy(a_v, a16_out.at[pl.ds(base, KG)])
        pltpu.sync_copy(b_v, b16_out.at[pl.ds(base, KG)])
        return 0

    lax.fori_loop(0, E_PT // KG, chunk, 0)


def _gather_sc(pos16, iidx, jidx):
    mesh = plsc.VectorSubcoreMesh(core_axis_name="c", subcore_axis_name="s",
                                  num_cores=NC, num_subcores=NS)
    f = pl.kernel(
        _gather_body,
        out_type=[
            jax.ShapeDtypeStruct((E, 16), jnp.float32),
            jax.ShapeDtypeStruct((E, 16), jnp.float32),
        ],
        mesh=mesh,
        compiler_params=_SC_PARAMS,
        scratch_types=[
            pltpu.VMEM((KG,), jnp.int32),
            pltpu.VMEM((KG,), jnp.int32),
            pltpu.VMEM((KG, 16), jnp.float32),
            pltpu.VMEM((KG, 16), jnp.float32),
        ],
    )
    return f(pos16, iidx, jidx)


def _scatter_body(ts16, iidx, mdu, mdw, kj, ji, jj, ft,
                  accs_out,
                  acc, idx_v, pay_v, kj_v, ji_v, jj_v, ft_v, u_v, w_v):
    c = lax.axis_index("c")
    s = lax.axis_index("s")
    wid = s * NC + c

    zero = jnp.zeros((16,), jnp.float32)

    def zrow(r, _):
        pay_v[r] = zero
        return 0

    lax.fori_loop(0, KS, zrow, 0)
    r0 = s * ROWS_PT
    done = 0
    while done < ROWS_PT:
        step = min(KS, ROWS_PT - done)
        pltpu.sync_copy(pay_v.at[pl.ds(0, step)], acc.at[pl.ds(r0 + done, step)])
        done += step
    plsc.subcore_barrier()

    def sym_chunk(k, _):
        base = wid * E_PT + k * KS
        pltpu.sync_copy(iidx.at[pl.ds(base, KS)], idx_v)
        pltpu.sync_copy(ts16.at[pl.ds(base, KS)], pay_v)
        pltpu.sync_copy(pay_v, acc.at[idx_v], add=True)
        return 0

    lax.fori_loop(0, E_PT // KS, sym_chunk, 0)

    def cross_chunk(k, _):
        base = wid * T_PT + k * KC
        pltpu.sync_copy(kj.at[pl.ds(base, KC)], kj_v)
        pltpu.sync_copy(ji.at[pl.ds(base, KC)], ji_v)
        pltpu.sync_copy(jj.at[pl.ds(base, KC)], jj_v)
        pltpu.sync_copy(ft.at[pl.ds(base, KC)], ft_v.at[pl.ds(0, KC)])
        pltpu.sync_copy(mdu.at[kj_v], u_v)
        pltpu.sync_copy(mdw.at[ji_v], w_v)

        def row(r, _):
            v = ft_v[pl.ds(r, 16)]
            pay_v[r] = u_v[r] * w_v[r] * jnp.full((16,), v[0], jnp.float32)
            return 0

        lax.fori_loop(0, KC, row, 0)
        pltpu.sync_copy(pay_v, acc.at[jj_v], add=True)
        return 0

    lax.fori_loop(0, T_PT // KC, cross_chunk, 0)

    plsc.subcore_barrier()
    pltpu.sync_copy(acc.at[pl.ds(s * ROWS_PT, ROWS_PT)],
                    accs_out.at[c, pl.ds(s * ROWS_PT, ROWS_PT)])


def _scatter_sc(ts16, iidx, mdu, mdw, kj, ji, jj, ft):
    mesh = plsc.VectorSubcoreMesh(core_axis_name="c", subcore_axis_name="s",
                                  num_cores=NC, num_subcores=NS)
    f = pl.kernel(
        _scatter_body,
        out_type=[jax.ShapeDtypeStruct((NC, NPAD, 16), jnp.float32)],
        mesh=mesh,
        compiler_params=_SC_PARAMS,
        scratch_types=[
            pltpu.VMEM_SHARED((NPAD, 16), jnp.float32),
            pltpu.VMEM((KS,), jnp.int32),
            pltpu.VMEM((KS, 16), jnp.float32),
            pltpu.VMEM((KC,), jnp.int32),
            pltpu.VMEM((KC,), jnp.int32),
            pltpu.VMEM((KC,), jnp.int32),
            pltpu.VMEM((KC + 16,), jnp.float32),
            pltpu.VMEM((KC, 16), jnp.float32),
            pltpu.VMEM((KC, 16), jnp.float32),
        ],
    )
    return f(ts16, iidx, mdu, mdw, kj, ji, jj, ft)[0]


def kernel(pos, edge_index, nbr_shift, edge_attr, triplet_attr, triplet_index,
           We0, be0, We1, be1, We2, be2, Weo, beo,
           Wt0, bt0, Wt1, bt1, Wt2, bt2, Wto, bto):
    We0b, We1b, We2b = (_block_diag(W, PACK) for W in (We0, We1, We2))
    Wt0b, Wt1b, Wt2b = (_block_diag(W, PACK) for W in (Wt0, Wt1, Wt2))
    Weob = _block_diag(Weo, PACK)
    Wtob = _block_diag(Wto, PACK)
    be0b, be1b, be2b = (jnp.tile(b, PACK)[None, :] for b in (be0, be1, be2))
    bt0b, bt1b, bt2b = (jnp.tile(b, PACK)[None, :] for b in (bt0, bt1, bt2))
    beob = jnp.tile(beo, PACK)[None, :]
    btob = jnp.tile(bto, PACK)[None, :]

    fe = _packed_mlp(edge_attr.reshape(E // PACK, DBIG),
                     We0b, be0b, We1b, be1b, We2b, be2b, Weob, beob,
                     rows_blk=1000).reshape(E, 1)
    ft = _packed_mlp(triplet_attr.reshape(T // PACK, DBIG),
                     Wt0b, bt0b, Wt1b, bt1b, Wt2b, bt2b, Wtob, btob,
                     rows_blk=1000).reshape(T)

    pos16 = jnp.pad(pos, ((0, 0), (0, 13)))
    j = edge_index[0]
    i = edge_index[1]
    a16, b16 = _gather_sc(pos16, i, j)

    sh4 = jnp.pad(nbr_shift, ((0, 0), (0, 1)))
    ts16, mdu, mdw = _geometry(a16, b16, sh4, fe)

    jj = triplet_index[1]
    kj = triplet_index[3]
    ji = triplet_index[4]
    accs = _scatter_sc(ts16, i, mdu, mdw, kj, ji, jj, ft)

    born16 = accs[0] + accs[1]
    return born16[:N, jnp.asarray(_DYAD_LANES)]


# speedup vs baseline: 12.7166x; 1.4823x over previous
import functools

import jax
import jax.numpy as jnp
import numpy as np
from jax import lax
from jax.experimental import pallas as pl
from jax.experimental.pallas import tpu as pltpu
from jax.experimental.pallas import tpu_sc as plsc

N = 50000
E = 800000
T = 1600000
D = 16
PACK = 16
DBIG = D * PACK
CUTOFF2 = 36.0

NC = 2
NS = 16
NW = NC * NS
NPAD = 50048
ROWS_PT = NPAD // NS

E_PT = E // NW
T_PT = T // NW
KG = 1000
KS = 1000
KC = 1000

_SC_PARAMS = pltpu.CompilerParams(use_tc_tiling_on_sc=False)

_PU = np.zeros((4, 16), np.float32)
_PW = np.zeros((4, 16), np.float32)
for _c in range(3):
    _PU[_c, 4 * _c:4 * _c + 4] = 1.0
    _PW[_c, _c::4] = 1.0
_DYAD_LANES = np.array([0, 1, 2, 4, 5, 6, 8, 9, 10], np.int32)


def _block_diag(W, pack):
    D_, Do = W.shape
    out = jnp.zeros((pack, D_, pack, Do), W.dtype)
    idx = jnp.arange(pack)
    out = out.at[idx, :, idx, :].set(jnp.broadcast_to(W, (pack, D_, Do)))
    return out.reshape(pack * D_, pack * Do)


def _softplus(x):
    return jnp.maximum(x, 0.0) + jnp.log1p(jnp.exp(-jnp.abs(x)))


def _mlp_body(x_ref, w0, b0, w1, b1, w2, b2, wo, bo, out_ref):
    h = x_ref[...]
    h = _softplus(jnp.dot(h, w0[...], preferred_element_type=jnp.float32) + b0[...])
    h = _softplus(jnp.dot(h, w1[...], preferred_element_type=jnp.float32) + b1[...])
    h = _softplus(jnp.dot(h, w2[...], preferred_element_type=jnp.float32) + b2[...])
    out_ref[...] = jnp.dot(h, wo[...], preferred_element_type=jnp.float32) + bo[...]


def _packed_mlp(x, W0, b0, W1, b1, W2, b2, Wo, bo, rows_blk):
    R = x.shape[0]
    grid = (R // rows_blk,)
    wspec16 = pl.BlockSpec((DBIG, DBIG), lambda i: (0, 0))
    wspec_o = pl.BlockSpec((DBIG, PACK), lambda i: (0, 0))
    bspec = pl.BlockSpec((1, DBIG), lambda i: (0, 0))
    bspec_o = pl.BlockSpec((1, PACK), lambda i: (0, 0))
    return pl.pallas_call(
        _mlp_body,
        grid=grid,
        in_specs=[
            pl.BlockSpec((rows_blk, DBIG), lambda i: (i, 0)),
            wspec16, bspec, wspec16, bspec, wspec16, bspec, wspec_o, bspec_o,
        ],
        out_specs=pl.BlockSpec((rows_blk, PACK), lambda i: (i, 0)),
        out_shape=jax.ShapeDtypeStruct((R, PACK), jnp.float32),
    )(x, W0, b0, W1, b1, W2, b2, Wo, bo)


def _geo_body(a, b, sh, fe8, pu, pw, qq, s8, ts, mdu, mdw):
    hp = lax.Precision.HIGHEST
    d = a[...] - b[...] + sh[...]
    d2 = jnp.dot(d * d, qq[...], preferred_element_type=jnp.float32, precision=hp)
    ir = lax.rsqrt(d2)
    u = jnp.dot(d, pu[...], preferred_element_type=jnp.float32, precision=hp)
    w = jnp.dot(d, pw[...], preferred_element_type=jnp.float32, precision=hp)
    fe = jnp.dot(fe8[...], s8[...], preferred_element_type=jnp.float32, precision=hp)
    ir2 = ir * ir
    ts[...] = fe * u * w * ir2
    m = jnp.where(d2 < CUTOFF2, ir, 0.0)
    mdu[...] = u * m
    mdw[...] = w * m


def _geometry(a128, b128, sh128, fe8, rows_blk=2000):
    E8 = E // 8
    grid = (E8 // rows_blk,)
    spec128 = pl.BlockSpec((rows_blk, 128), lambda i: (i, 0))
    spec8 = pl.BlockSpec((rows_blk, 8), lambda i: (i, 0))
    specp = pl.BlockSpec((128, 128), lambda i: (0, 0))
    specs8 = pl.BlockSpec((8, 128), lambda i: (0, 0))
    shp = jax.ShapeDtypeStruct((E8, 128), jnp.float32)

    def bd(m16):
        out = np.zeros((128, 128), np.float32)
        for k in range(8):
            out[16 * k:16 * k + 16, 16 * k:16 * k + 16] = m16
        return jnp.asarray(out)

    pu16 = np.zeros((16, 16), np.float32)
    pw16 = np.zeros((16, 16), np.float32)
    for c in range(3):
        pu16[c, 4 * c:4 * c + 4] = 1.0
        pw16[c, c::4] = 1.0
    q16 = np.zeros((16, 16), np.float32)
    q16[0:4, :] = 1.0
    s8 = np.zeros((8, 128), np.float32)
    for e in range(8):
        s8[e, 16 * e:16 * e + 16] = 1.0

    return pl.pallas_call(
        _geo_body,
        grid=grid,
        in_specs=[spec128, spec128, spec128, spec8, specp, specp, specp, specs8],
        out_specs=[spec128, spec128, spec128],
        out_shape=[shp, shp, shp],
    )(a128, b128, sh128, fe8, bd(pu16), bd(pw16), bd(q16), jnp.asarray(s8))


def _gather_body(pos16, iidx, jidx, a16_out, b16_out, i_v, j_v, a_v, b_v):
    c = lax.axis_index("c")
    s = lax.axis_index("s")
    wid = s * NC + c

    def chunk(k, _):
        base = wid * E_PT + k * KG
        pltpu.sync_copy(iidx.at[pl.ds(base, KG)], i_v)
        pltpu.sync_copy(jidx.at[pl.ds(base, KG)], j_v)
        pltpu.sync_copy(pos16.at[i_v], a_v)
        pltpu.sync_copy(pos16.at[j_v], b_v)
        pltpu.sync_copy(a_v, a16_out.at[pl.ds(base, KG)])
        pltpu.sync_copy(b_v, b16_out.at[pl.ds(base, KG)])
        return 0

    lax.fori_loop(0, E_PT // KG, chunk, 0)


def _gather_sc(pos16, iidx, jidx):
    mesh = plsc.VectorSubcoreMesh(core_axis_name="c", subcore_axis_name="s",
                                  num_cores=NC, num_subcores=NS)
    f = pl.kernel(
        _gather_body,
        out_type=[
            jax.ShapeDtypeStruct((E, 16), jnp.float32),
            jax.ShapeDtypeStruct((E, 16), jnp.float32),
        ],
        mesh=mesh,
        compiler_params=_SC_PARAMS,
        scratch_types=[
            pltpu.VMEM((KG,), jnp.int32),
            pltpu.VMEM((KG,), jnp.int32),
            pltpu.VMEM((KG, 16), jnp.float32),
            pltpu.VMEM((KG, 16), jnp.float32),
        ],
    )
    return f(pos16, iidx, jidx)


def _scatter_body(ts16, iidx, mdu, mdw, kj, ji, jj, ft,
                  accs_out,
                  acc, idx_v, pay_v, kj_v, ji_v, jj_v, ft_v, u_v, w_v):
    c = lax.axis_index("c")
    s = lax.axis_index("s")
    wid = s * NC + c

    zero = jnp.zeros((16,), jnp.float32)

    def zrow(r, _):
        pay_v[r] = zero
        return 0

    lax.fori_loop(0, KS, zrow, 0)
    r0 = s * ROWS_PT
    done = 0
    while done < ROWS_PT:
        step = min(KS, ROWS_PT - done)
        pltpu.sync_copy(pay_v.at[pl.ds(0, step)], acc.at[pl.ds(r0 + done, step)])
        done += step
    plsc.subcore_barrier()

    def sym_chunk(k, _):
        base = wid * E_PT + k * KS
        pltpu.sync_copy(iidx.at[pl.ds(base, KS)], idx_v)
        pltpu.sync_copy(ts16.at[pl.ds(base, KS)], pay_v)
        pltpu.sync_copy(pay_v, acc.at[idx_v], add=True)
        return 0

    lax.fori_loop(0, E_PT // KS, sym_chunk, 0)

    def cross_chunk(k, _):
        base = wid * T_PT + k * KC
        pltpu.sync_copy(kj.at[pl.ds(base, KC)], kj_v)
        pltpu.sync_copy(ji.at[pl.ds(base, KC)], ji_v)
        pltpu.sync_copy(jj.at[pl.ds(base, KC)], jj_v)
        pltpu.sync_copy(ft.at[pl.ds(base, KC)], ft_v.at[pl.ds(0, KC)])
        pltpu.sync_copy(mdu.at[kj_v], u_v)
        pltpu.sync_copy(mdw.at[ji_v], w_v)

        def row(r, _):
            v = ft_v[pl.ds(r, 16)]
            pay_v[r] = u_v[r] * w_v[r] * jnp.full((16,), v[0], jnp.float32)
            return 0

        lax.fori_loop(0, KC, row, 0)
        pltpu.sync_copy(pay_v, acc.at[jj_v], add=True)
        return 0

    lax.fori_loop(0, T_PT // KC, cross_chunk, 0)

    plsc.subcore_barrier()
    pltpu.sync_copy(acc.at[pl.ds(s * ROWS_PT, ROWS_PT)],
                    accs_out.at[c, pl.ds(s * ROWS_PT, ROWS_PT)])


def _scatter_sc(ts16, iidx, mdu, mdw, kj, ji, jj, ft):
    mesh = plsc.VectorSubcoreMesh(core_axis_name="c", subcore_axis_name="s",
                                  num_cores=NC, num_subcores=NS)
    f = pl.kernel(
        _scatter_body,
        out_type=[jax.ShapeDtypeStruct((NC, NPAD, 16), jnp.float32)],
        mesh=mesh,
        compiler_params=_SC_PARAMS,
        scratch_types=[
            pltpu.VMEM_SHARED((NPAD, 16), jnp.float32),
            pltpu.VMEM((KS,), jnp.int32),
            pltpu.VMEM((KS, 16), jnp.float32),
            pltpu.VMEM((KC,), jnp.int32),
            pltpu.VMEM((KC,), jnp.int32),
            pltpu.VMEM((KC,), jnp.int32),
            pltpu.VMEM((KC + 16,), jnp.float32),
            pltpu.VMEM((KC, 16), jnp.float32),
            pltpu.VMEM((KC, 16), jnp.float32),
        ],
    )
    return f(ts16, iidx, mdu, mdw, kj, ji, jj, ft)[0]


def kernel(pos, edge_index, nbr_shift, edge_attr, triplet_attr, triplet_index,
           We0, be0, We1, be1, We2, be2, Weo, beo,
           Wt0, bt0, Wt1, bt1, Wt2, bt2, Wto, bto):
    We0b, We1b, We2b = (_block_diag(W, PACK) for W in (We0, We1, We2))
    Wt0b, Wt1b, Wt2b = (_block_diag(W, PACK) for W in (Wt0, Wt1, Wt2))
    Weob = _block_diag(Weo, PACK)
    Wtob = _block_diag(Wto, PACK)
    be0b, be1b, be2b = (jnp.tile(b, PACK)[None, :] for b in (be0, be1, be2))
    bt0b, bt1b, bt2b = (jnp.tile(b, PACK)[None, :] for b in (bt0, bt1, bt2))
    beob = jnp.tile(beo, PACK)[None, :]
    btob = jnp.tile(bto, PACK)[None, :]

    xe = edge_attr.T.reshape(D, E // PACK, PACK).transpose(1, 2, 0).reshape(E // PACK, DBIG)
    xt = triplet_attr.T.reshape(D, T // PACK, PACK).transpose(1, 2, 0).reshape(T // PACK, DBIG)
    fe = _packed_mlp(xe,
                     We0b, be0b, We1b, be1b, We2b, be2b, Weob, beob,
                     rows_blk=1000).reshape(E // 8, 8)
    ft = _packed_mlp(xt,
                     Wt0b, bt0b, Wt1b, bt1b, Wt2b, bt2b, Wtob, btob,
                     rows_blk=1000).reshape(T)

    pos16 = jnp.pad(pos, ((0, 0), (0, 13)))
    j = edge_index[0]
    i = edge_index[1]
    a16, b16 = _gather_sc(pos16, i, j)

    sh128 = jnp.pad(nbr_shift, ((0, 0), (0, 13))).reshape(E // 8, 128)
    a128 = a16.reshape(E // 8, 128)
    b128 = b16.reshape(E // 8, 128)
    ts128, mdu128, mdw128 = _geometry(a128, b128, sh128, fe)
    ts16 = ts128.reshape(E, 16)
    mdu = mdu128.reshape(E, 16)
    mdw = mdw128.reshape(E, 16)

    jj = triplet_index[1]
    kj = triplet_index[3]
    ji = triplet_index[4]
    accs = _scatter_sc(ts16, i, mdu, mdw, kj, ji, jj, ft)

    born16 = accs[0] + accs[1]
    return born16[:N, jnp.asarray(_DYAD_LANES)]


# speedup vs baseline: 16.4159x; 1.2909x over previous
import functools

import jax
import jax.numpy as jnp
import numpy as np
from jax import lax
from jax.experimental import pallas as pl
from jax.experimental.pallas import tpu as pltpu
from jax.experimental.pallas import tpu_sc as plsc

N = 50000
E = 800000
T = 1600000
D = 16
PACK = 16
DBIG = D * PACK
CUTOFF2 = 36.0

NC = 2
NS = 16
NW = NC * NS
NPAD = 50048
ROWS_PT = NPAD // NS

E_PT = E // NW
T_PT = T // NW
KG = 1000
KS = 1000
KC = 1000

_SC_PARAMS = pltpu.CompilerParams(use_tc_tiling_on_sc=False)

_PU = np.zeros((4, 16), np.float32)
_PW = np.zeros((4, 16), np.float32)
for _c in range(3):
    _PU[_c, 4 * _c:4 * _c + 4] = 1.0
    _PW[_c, _c::4] = 1.0
_DYAD_LANES = np.array([0, 1, 2, 4, 5, 6, 8, 9, 10], np.int32)


def _mlp_t_body(x_ref, w0, b0, w1, b1, w2, b2, wo, bo, out_ref):
    h = x_ref[...]
    h = _softplus(jnp.dot(w0[...], h, preferred_element_type=jnp.float32) + b0[...])
    h = _softplus(jnp.dot(w1[...], h, preferred_element_type=jnp.float32) + b1[...])
    h = _softplus(jnp.dot(w2[...], h, preferred_element_type=jnp.float32) + b2[...])
    o = jnp.dot(wo[...], h, preferred_element_type=jnp.float32) + bo[...]
    out_ref[...] = o.reshape(o.shape[1])


def _softplus(x):
    return jnp.maximum(x, 0.0) + jnp.log1p(jnp.exp(-jnp.abs(x)))


def _mlp_t(xT, W0, b0, W1, b1, W2, b2, Wo, bo, cols_blk=5120):
    R = xT.shape[1]
    grid = (pl.cdiv(R, cols_blk),)
    wspec = pl.BlockSpec((D, D), lambda i: (0, 0))
    bspec = pl.BlockSpec((D, 1), lambda i: (0, 0))
    wospec = pl.BlockSpec((1, D), lambda i: (0, 0))
    bospec = pl.BlockSpec((1, 1), lambda i: (0, 0))
    return pl.pallas_call(
        _mlp_t_body,
        grid=grid,
        in_specs=[
            pl.BlockSpec((D, cols_blk), lambda i: (0, i)),
            wspec, bspec, wspec, bspec, wspec, bspec, wospec, bospec,
        ],
        out_specs=pl.BlockSpec((cols_blk,), lambda i: (i,)),
        out_shape=jax.ShapeDtypeStruct((R,), jnp.float32),
    )(xT, W0.T, b0[:, None], W1.T, b1[:, None], W2.T, b2[:, None],
      Wo.T, bo[:, None])


def _geo_body(a, b, sh, fe128, pu, pw, qq, ts, mdu, mdw):
    hp = lax.Precision.HIGHEST
    d = a[...] - b[...] + sh[...]
    d2 = jnp.dot(d * d, qq[...], preferred_element_type=jnp.float32, precision=hp)
    ir = lax.rsqrt(d2)
    u = jnp.dot(d, pu[...], preferred_element_type=jnp.float32, precision=hp)
    w = jnp.dot(d, pw[...], preferred_element_type=jnp.float32, precision=hp)
    fe = fe128[...]
    ir2 = ir * ir
    ts[...] = fe * u * w * ir2
    m = jnp.where(d2 < CUTOFF2, ir, 0.0)
    mdu[...] = u * m
    mdw[...] = w * m


def _geometry(a128, b128, sh128, fe128, rows_blk=2000):
    E8 = E // 8
    grid = (E8 // rows_blk,)
    spec128 = pl.BlockSpec((rows_blk, 128), lambda i: (i, 0))
    specp = pl.BlockSpec((128, 128), lambda i: (0, 0))
    shp = jax.ShapeDtypeStruct((E8, 128), jnp.float32)

    def bd(m16):
        out = np.zeros((128, 128), np.float32)
        for k in range(8):
            out[16 * k:16 * k + 16, 16 * k:16 * k + 16] = m16
        return jnp.asarray(out)

    pu16 = np.zeros((16, 16), np.float32)
    pw16 = np.zeros((16, 16), np.float32)
    for c in range(3):
        pu16[c, 4 * c:4 * c + 4] = 1.0
        pw16[c, c::4] = 1.0
    q16 = np.zeros((16, 16), np.float32)
    q16[0:4, :] = 1.0
    return pl.pallas_call(
        _geo_body,
        grid=grid,
        in_specs=[spec128, spec128, spec128, spec128, specp, specp, specp],
        out_specs=[spec128, spec128, spec128],
        out_shape=[shp, shp, shp],
    )(a128, b128, sh128, fe128, bd(pu16), bd(pw16), bd(q16))


def _gather_body(pos16, iidx, jidx, a16_out, b16_out, i_v, j_v, a_v, b_v):
    c = lax.axis_index("c")
    s = lax.axis_index("s")
    wid = s * NC + c

    def chunk(k, _):
        base = wid * E_PT + k * KG
        pltpu.sync_copy(iidx.at[pl.ds(base, KG)], i_v)
        pltpu.sync_copy(jidx.at[pl.ds(base, KG)], j_v)
        pltpu.sync_copy(pos16.at[i_v], a_v)
        pltpu.sync_copy(pos16.at[j_v], b_v)
        pltpu.sync_copy(a_v, a16_out.at[pl.ds(base, KG)])
        pltpu.sync_copy(b_v, b16_out.at[pl.ds(base, KG)])
        return 0

    lax.fori_loop(0, E_PT // KG, chunk, 0)


def _gather_sc(pos16, iidx, jidx):
    mesh = plsc.VectorSubcoreMesh(core_axis_name="c", subcore_axis_name="s",
                                  num_cores=NC, num_subcores=NS)
    f = pl.kernel(
        _gather_body,
        out_type=[
            jax.ShapeDtypeStruct((E, 16), jnp.float32),
            jax.ShapeDtypeStruct((E, 16), jnp.float32),
        ],
        mesh=mesh,
        compiler_params=_SC_PARAMS,
        scratch_types=[
            pltpu.VMEM((KG,), jnp.int32),
            pltpu.VMEM((KG,), jnp.int32),
            pltpu.VMEM((KG, 16), jnp.float32),
            pltpu.VMEM((KG, 16), jnp.float32),
        ],
    )
    return f(pos16, iidx, jidx)


def _scatter_body(ts16, iidx, mdu, mdw, kj, ji, jj, ft,
                  accs_out,
                  acc, idx_v, pay_v, kj_v, ji_v, jj_v, ft_v, u_v, w_v):
    c = lax.axis_index("c")
    s = lax.axis_index("s")
    wid = s * NC + c

    zero = jnp.zeros((16,), jnp.float32)

    def zrow(r, _):
        pay_v[r] = zero
        return 0

    lax.fori_loop(0, KS, zrow, 0)
    r0 = s * ROWS_PT
    done = 0
    while done < ROWS_PT:
        step = min(KS, ROWS_PT - done)
        pltpu.sync_copy(pay_v.at[pl.ds(0, step)], acc.at[pl.ds(r0 + done, step)])
        done += step
    plsc.subcore_barrier()

    def sym_chunk(k, _):
        base = wid * E_PT + k * KS
        pltpu.sync_copy(iidx.at[pl.ds(base, KS)], idx_v)
        pltpu.sync_copy(ts16.at[pl.ds(base, KS)], pay_v)
        pltpu.sync_copy(pay_v, acc.at[idx_v], add=True)
        return 0

    lax.fori_loop(0, E_PT // KS, sym_chunk, 0)

    def cross_chunk(k, _):
        base = wid * T_PT + k * KC
        pltpu.sync_copy(kj.at[pl.ds(base, KC)], kj_v)
        pltpu.sync_copy(ji.at[pl.ds(base, KC)], ji_v)
        pltpu.sync_copy(jj.at[pl.ds(base, KC)], jj_v)
        pltpu.sync_copy(ft.at[pl.ds(base, KC)], ft_v.at[pl.ds(0, KC)])
        pltpu.sync_copy(mdu.at[kj_v], u_v)
        pltpu.sync_copy(mdw.at[ji_v], w_v)

        def row(r, _):
            v = ft_v[pl.ds(r, 16)]
            pay_v[r] = u_v[r] * w_v[r] * jnp.full((16,), v[0], jnp.float32)
            return 0

        lax.fori_loop(0, KC, row, 0)
        pltpu.sync_copy(pay_v, acc.at[jj_v], add=True)
        return 0

    lax.fori_loop(0, T_PT // KC, cross_chunk, 0)

    plsc.subcore_barrier()
    pltpu.sync_copy(acc.at[pl.ds(s * ROWS_PT, ROWS_PT)],
                    accs_out.at[c, pl.ds(s * ROWS_PT, ROWS_PT)])


def _scatter_sc(ts16, iidx, mdu, mdw, kj, ji, jj, ft):
    mesh = plsc.VectorSubcoreMesh(core_axis_name="c", subcore_axis_name="s",
                                  num_cores=NC, num_subcores=NS)
    f = pl.kernel(
        _scatter_body,
        out_type=[jax.ShapeDtypeStruct((NC, NPAD, 16), jnp.float32)],
        mesh=mesh,
        compiler_params=_SC_PARAMS,
        scratch_types=[
            pltpu.VMEM_SHARED((NPAD, 16), jnp.float32),
            pltpu.VMEM((KS,), jnp.int32),
            pltpu.VMEM((KS, 16), jnp.float32),
            pltpu.VMEM((KC,), jnp.int32),
            pltpu.VMEM((KC,), jnp.int32),
            pltpu.VMEM((KC,), jnp.int32),
            pltpu.VMEM((KC + 16,), jnp.float32),
            pltpu.VMEM((KC, 16), jnp.float32),
            pltpu.VMEM((KC, 16), jnp.float32),
        ],
    )
    return f(ts16, iidx, mdu, mdw, kj, ji, jj, ft)[0]


def kernel(pos, edge_index, nbr_shift, edge_attr, triplet_attr, triplet_index,
           We0, be0, We1, be1, We2, be2, Weo, beo,
           Wt0, bt0, Wt1, bt1, Wt2, bt2, Wto, bto):
    fe = _mlp_t(edge_attr.T, We0, be0, We1, be1, We2, be2, Weo, beo)
    ft = _mlp_t(triplet_attr.T, Wt0, bt0, Wt1, bt1, Wt2, bt2, Wto, bto)

    pos16 = jnp.pad(pos, ((0, 0), (0, 13)))
    j = edge_index[0]
    i = edge_index[1]
    a16, b16 = _gather_sc(pos16, i, j)

    sh128 = jnp.pad(nbr_shift, ((0, 0), (0, 13))).reshape(E // 8, 128)
    a128 = a16.reshape(E // 8, 128)
    b128 = b16.reshape(E // 8, 128)
    fe128 = jnp.repeat(fe, 16).reshape(E // 8, 128)
    ts128, mdu128, mdw128 = _geometry(a128, b128, sh128, fe128)
    ts16 = ts128.reshape(E, 16)
    mdu = mdu128.reshape(E, 16)
    mdw = mdw128.reshape(E, 16)

    jj = triplet_index[1]
    kj = triplet_index[3]
    ji = triplet_index[4]
    accs = _scatter_sc(ts16, i, mdu, mdw, kj, ji, jj, ft)

    born16 = accs[0] + accs[1]
    return born16[:N, jnp.asarray(_DYAD_LANES)]
